# baseline (speedup 1.0000x reference)
import functools

import jax
import jax.numpy as jnp
from jax import lax
from jax.experimental import pallas as pl
from jax.experimental.pallas import tpu as pltpu
from jax.experimental.pallas import tpu_sc as plsc

N_CLASSES = 1000
BATCH = 16384

NC = 2
NS = 16
L = 16
NW = NC * NS
W_BIG = 29
TR_BIG, TR_SML = 4, 3
Q = 4
C_Q = BATCH // Q
GROUPS = C_Q // L
UNROLL = 8

_mesh = plsc.VectorSubcoreMesh(core_axis_name="c", subcore_axis_name="s")


@functools.partial(
    pl.kernel,
    out_type=jax.ShapeDtypeStruct((N_CLASSES, BATCH), jnp.float32),
    mesh=_mesh,
    scratch_types=[
        pltpu.VMEM((C_Q,), jnp.int32),
        pltpu.VMEM((C_Q,), jnp.int32),
        pltpu.VMEM((8, C_Q), jnp.float32),
        pltpu.VMEM((8, C_Q), jnp.float32),
        pltpu.SemaphoreType.DMA,
        pltpu.SemaphoreType.DMA,
    ],
    compiler_params=pltpu.CompilerParams(needs_layout_passes=False),
)
def _one_hot_t_sc(idx_hbm, z_hbm, out_hbm, idx0, idx1, buf0, buf1,
                  sem0, sem1):
    wid = lax.axis_index("s") * NC + lax.axis_index("c")

    pltpu.sync_copy(z_hbm, buf0)
    pltpu.sync_copy(z_hbm, buf1)

    zeros16 = jnp.zeros((L,), jnp.float32)
    ones16 = jnp.ones((L,), jnp.float32)
    lane = lax.iota(jnp.int32, L)
    idxbufs = (idx0, idx1)
    bufs = (buf0, buf1)
    sems = (sem0, sem1)

    def _scan(idxbuf, buf, row0, vals):
        def grp(i, carry):
            for u in range(UNROLL):
                g = i * UNROLL + u
                idxv = idxbuf[pl.ds(g * L, L)]
                rel = idxv - row0
                mask = (rel >= 0) & (rel < 8)
                plsc.store_scatter(buf, (rel, lane + g * L), vals,
                                   mask=mask)
            return carry

        lax.fori_loop(0, GROUPS // UNROLL, grp, 0)

    def _wait(buf, sem):
        pltpu.make_async_copy(
            buf, out_hbm.at[pl.ds(0, 8), pl.ds(0, C_Q)], sem).wait()

    def _run(row0, ntr):
        pend = [None, None]
        k = 0
        for q in range(Q):
            qb = idxbufs[q % 2]
            pltpu.sync_copy(idx_hbm.at[pl.ds(q * C_Q, C_Q)], qb)
            for t in range(ntr):
                b = k % 2
                buf, sem = bufs[b], sems[b]
                if pend[b] is not None:
                    _wait(buf, sem)
                    _scan(pend[b][0], buf, pend[b][1], zeros16)
                rb = row0 + 8 * t
                _scan(qb, buf, rb, ones16)
                pltpu.async_copy(
                    buf, out_hbm.at[pl.ds(rb, 8), pl.ds(q * C_Q, C_Q)],
                    sem)
                pend[b] = (qb, rb)
                k += 1
        _wait(bufs[0], sems[0])
        _wait(bufs[1], sems[1])

    @pl.when(wid < W_BIG)
    def _():
        _run(32 * wid, TR_BIG)

    @pl.when(wid >= W_BIG)
    def _():
        _run(8 * TR_BIG * W_BIG + 24 * (wid - W_BIG), TR_SML)


def kernel(inputs):
    idx = inputs.astype(jnp.int32)
    zblk = jnp.zeros((8, C_Q), jnp.float32)
    out_t = _one_hot_t_sc(idx, zblk)
    return out_t.T


# speedup vs baseline: 1.7177x; 1.7177x over previous
import functools

import jax
import jax.numpy as jnp
from jax import lax
from jax.experimental import pallas as pl
from jax.experimental.pallas import tpu as pltpu
from jax.experimental.pallas import tpu_sc as plsc

N_CLASSES = 1000
BATCH = 16384

NC = 2
NS = 16
L = 16
NW = NC * NS
COLS_PER_W = BATCH // NW
R_BLK = 120
N_BLKS = 9
R_TAIL = N_CLASSES - (N_BLKS - 1) * R_BLK

_mesh = plsc.VectorSubcoreMesh(core_axis_name="c", subcore_axis_name="s")


@functools.partial(
    pl.kernel,
    out_type=jax.ShapeDtypeStruct((N_CLASSES, BATCH), jnp.float32),
    mesh=_mesh,
    scratch_types=[
        pltpu.VMEM((COLS_PER_W,), jnp.int32),
        pltpu.VMEM((R_BLK, COLS_PER_W), jnp.float32),
        pltpu.VMEM((R_BLK, COLS_PER_W), jnp.float32),
        pltpu.SemaphoreType.DMA,
        pltpu.SemaphoreType.DMA,
    ],
    compiler_params=pltpu.CompilerParams(needs_layout_passes=False),
)
def _one_hot_t_sc(idx_hbm, z_hbm, out_hbm, idx_v, buf0, buf1, sem0, sem1):
    wid = lax.axis_index("s") * NC + lax.axis_index("c")
    col0 = wid * COLS_PER_W

    pltpu.sync_copy(idx_hbm.at[pl.ds(col0, COLS_PER_W)], idx_v)
    pltpu.sync_copy(z_hbm, buf0)
    pltpu.sync_copy(z_hbm, buf1)

    zeros16 = jnp.zeros((L,), jnp.float32)
    ones16 = jnp.ones((L,), jnp.float32)
    lane = lax.iota(jnp.int32, L)

    def _rows(blk):
        return R_TAIL if blk == N_BLKS - 1 else R_BLK

    def _scan(buf, blk, vals):
        nr = _rows(blk)
        for g in range(COLS_PER_W // L):
            idxv = idx_v[pl.ds(g * L, L)]
            rel = idxv - (blk * R_BLK)
            mask = (rel >= 0) & (rel < nr)
            plsc.store_scatter(buf, (rel, lane + g * L), vals, mask=mask)

    def _wait(buf, sem, blk):
        nr = _rows(blk)
        pltpu.make_async_copy(
            buf.at[pl.ds(0, nr)],
            out_hbm.at[pl.ds(0, nr), pl.ds(0, COLS_PER_W)], sem).wait()

    bufs = (buf0, buf1)
    sems = (sem0, sem1)
    pend = [None, None]
    for c in range(N_BLKS):
        b = c % 2
        buf, sem = bufs[b], sems[b]
        if pend[b] is not None:
            _wait(buf, sem, pend[b])
            _scan(buf, pend[b], zeros16)
        _scan(buf, c, ones16)
        nr = _rows(c)
        pltpu.async_copy(
            buf.at[pl.ds(0, nr)],
            out_hbm.at[pl.ds(c * R_BLK, nr), pl.ds(col0, COLS_PER_W)], sem)
        pend[b] = c
    _wait(bufs[0], sems[0], pend[0])
    _wait(bufs[1], sems[1], pend[1])


def kernel(inputs):
    idx = inputs.astype(jnp.int32)
    zblk = jnp.zeros((R_BLK, COLS_PER_W), jnp.float32)
    out_t = _one_hot_t_sc(idx, zblk)
    return out_t.T


# speedup vs baseline: 2.0666x; 1.2031x over previous
import functools

import jax
import jax.numpy as jnp
from jax import lax
from jax.experimental import pallas as pl
from jax.experimental.pallas import tpu as pltpu
from jax.experimental.pallas import tpu_sc as plsc

N_CLASSES = 1000
BATCH = 16384

NC = 2
NS = 16
L = 16
NW = NC * NS
COLS_PER_W = BATCH // NW
C_BLK = 128
N_BLKS = COLS_PER_W // C_BLK

_mesh = plsc.VectorSubcoreMesh(core_axis_name="c", subcore_axis_name="s")


@functools.partial(
    pl.kernel,
    out_type=jax.ShapeDtypeStruct((N_CLASSES, BATCH), jnp.float32),
    mesh=_mesh,
    scratch_types=[
        pltpu.VMEM((COLS_PER_W,), jnp.int32),
        pltpu.VMEM((N_CLASSES, C_BLK), jnp.float32),
    ],
    compiler_params=pltpu.CompilerParams(needs_layout_passes=False),
)
def _one_hot_t_sc(idx_hbm, z_hbm, out_hbm, idx_v, buf):
    wid = lax.axis_index("s") * NC + lax.axis_index("c")
    col0 = wid * COLS_PER_W

    pltpu.sync_copy(idx_hbm.at[pl.ds(col0, COLS_PER_W)], idx_v)
    pltpu.sync_copy(z_hbm, buf)

    zeros16 = jnp.zeros((L,), jnp.float32)
    ones16 = jnp.ones((L,), jnp.float32)
    lane = lax.iota(jnp.int32, L)

    def _flip(blk, vals):
        for g in range(C_BLK // L):
            idxv = idx_v[pl.ds(blk * C_BLK + g * L, L)]
            plsc.store_scatter(buf, (idxv, lane + (g * L)), vals)

    for c in range(N_BLKS):
        _flip(c, ones16)
        pltpu.sync_copy(buf, out_hbm.at[:, pl.ds(col0 + c * C_BLK, C_BLK)])
        if c + 1 < N_BLKS:
            _flip(c, zeros16)


def kernel(inputs):
    idx = inputs.astype(jnp.int32)
    zblk = jnp.zeros((N_CLASSES, C_BLK), jnp.float32)
    out_t = _one_hot_t_sc(idx, zblk)
    return out_t.T
